# Initial kernel scaffold; baseline (speedup 1.0000x reference)
import jax
import jax.numpy as jnp
from jax.experimental import pallas as pl


def kernel(x, table):
    raise NotImplementedError("write your pallas kernel here")


# speedup vs baseline: 1.6862x; 1.6862x over previous
import jax
import jax.numpy as jnp
from jax import lax
from jax.experimental import pallas as pl
from jax.experimental.pallas import tpu as pltpu
from jax.experimental.pallas import tpu_sc as plsc

_NC = 2
_NS = 16
_NW = _NC * _NS
_CHUNK = 128
_D = 64


def _body(x_hbm, table_hbm, out_hbm, idx_v, rows_v, gsem):
    nchunk = x_hbm.shape[1]
    per_w = nchunk * _CHUNK
    wid = lax.axis_index("s") * _NC + lax.axis_index("c")
    pltpu.sync_copy(x_hbm.at[wid], idx_v)
    base = wid * per_w

    def step(j, carry):
        pltpu.async_copy(table_hbm.at[idx_v.at[j]], rows_v, gsem).wait()
        pltpu.sync_copy(rows_v, out_hbm.at[pl.ds(base + j * _CHUNK, _CHUNK)])
        return carry

    lax.fori_loop(0, nchunk, step, 0)


def kernel(x, table):
    b, s = x.shape
    flat = b * s
    nchunk = flat // (_NW * _CHUNK)
    xf = x.reshape(_NW, nchunk, _CHUNK).astype(jnp.int32)
    mesh = plsc.VectorSubcoreMesh(core_axis_name="c", subcore_axis_name="s")
    out = pl.kernel(
        _body,
        out_type=jax.ShapeDtypeStruct((flat, _D), jnp.float32),
        mesh=mesh,
        scratch_types=[
            pltpu.VMEM((nchunk, _CHUNK), jnp.int32),
            pltpu.VMEM((_CHUNK, _D), jnp.float32),
            pltpu.SemaphoreType.DMA,
        ],
        compiler_params=pltpu.CompilerParams(use_tc_tiling_on_sc=False),
    )(xf, table)
    return out.reshape(b, s, _D)


# speedup vs baseline: 1.8761x; 1.1126x over previous
import jax
import jax.numpy as jnp
from jax import lax
from jax.experimental import pallas as pl
from jax.experimental.pallas import tpu as pltpu
from jax.experimental.pallas import tpu_sc as plsc

_NC = 2
_NS = 16
_NW = _NC * _NS
_CHUNK = 128
_D = 64


_NBUF = 4
_AHEAD = 2


def _body(x_hbm, table_hbm, out_hbm, idx_v, rows_v, gsem, ssem):
    nchunk = x_hbm.shape[1]
    per_w = nchunk * _CHUNK
    wid = lax.axis_index("s") * _NC + lax.axis_index("c")
    pltpu.sync_copy(x_hbm.at[wid], idx_v)
    base = wid * per_w

    def g_desc(j):
        return pltpu.make_async_copy(
            table_hbm.at[idx_v.at[j]], rows_v.at[j % _NBUF], gsem)

    def s_desc(j):
        return pltpu.make_async_copy(
            rows_v.at[j % _NBUF],
            out_hbm.at[pl.ds(base + j * _CHUNK, _CHUNK)], ssem)

    for j in range(_AHEAD):
        g_desc(j).start()

    def step(j, carry):
        @pl.when(j >= _AHEAD)
        def _():
            s_desc(j - _AHEAD).wait()

        @pl.when(j + _AHEAD < nchunk)
        def _():
            g_desc(j + _AHEAD).start()

        g_desc(j).wait()
        s_desc(j).start()
        return carry

    lax.fori_loop(0, nchunk, step, 0)
    for j in range(nchunk - _AHEAD, nchunk):
        s_desc(j).wait()


def kernel(x, table):
    b, s = x.shape
    flat = b * s
    nchunk = flat // (_NW * _CHUNK)
    xf = x.reshape(_NW, nchunk, _CHUNK).astype(jnp.int32)
    mesh = plsc.VectorSubcoreMesh(core_axis_name="c", subcore_axis_name="s")
    out = pl.kernel(
        _body,
        out_type=jax.ShapeDtypeStruct((flat, _D), jnp.float32),
        mesh=mesh,
        scratch_types=[
            pltpu.VMEM((nchunk, _CHUNK), jnp.int32),
            pltpu.VMEM((_NBUF, _CHUNK, _D), jnp.float32),
            pltpu.SemaphoreType.DMA,
            pltpu.SemaphoreType.DMA,
        ],
        compiler_params=pltpu.CompilerParams(use_tc_tiling_on_sc=False),
    )(xf, table)
    return out.reshape(b, s, _D)
